# baseline (speedup 1.0000x reference)
import functools

import jax
import jax.numpy as jnp
from jax import lax
from jax.experimental import pallas as pl
from jax.experimental.pallas import tpu as pltpu
from jax.experimental.pallas import tpu_sc as plsc

E = 16
K = 2
T = 128
D = 1024
F = 2048

_NC = 2
_LANES = 16
_CHUNKS = T // _LANES


def _combine_sc(r0_hbm, r1_hbm, w0_hbm, w1_hbm, out_hbm,
                r0_v, r1_v, w0_v, w1_v, acc_v):
    wid = lax.axis_index("s") * _NC + lax.axis_index("c")

    @pl.when(wid < _CHUNKS)
    def _():
        base = wid * _LANES
        pltpu.sync_copy(r0_hbm.at[pl.ds(base, _LANES)], r0_v)
        pltpu.sync_copy(r1_hbm.at[pl.ds(base, _LANES)], r1_v)
        pltpu.sync_copy(w0_hbm.at[pl.ds(base, _LANES)], w0_v)
        pltpu.sync_copy(w1_hbm.at[pl.ds(base, _LANES)], w1_v)
        for r in range(_LANES):
            acc_v[pl.ds(r * _LANES, _LANES)] = jnp.zeros((_LANES,), jnp.float32)
        t_off = lax.iota(jnp.int32, _LANES) * E
        plsc.addupdate_scatter(acc_v, [t_off + r0_v[...]], w0_v[...])
        plsc.addupdate_scatter(acc_v, [t_off + r1_v[...]], w1_v[...])
        pltpu.sync_copy(acc_v, out_hbm.at[pl.ds(base * E, _LANES * E)])


def _routing_combine(route, router_weights):
    sc = functools.partial(
        pl.kernel,
        out_type=jax.ShapeDtypeStruct((T * E,), jnp.float32),
        mesh=plsc.VectorSubcoreMesh(core_axis_name="c", subcore_axis_name="s"),
        scratch_types=[
            pltpu.VMEM((_LANES,), jnp.int32),
            pltpu.VMEM((_LANES,), jnp.int32),
            pltpu.VMEM((_LANES,), jnp.float32),
            pltpu.VMEM((_LANES,), jnp.float32),
            pltpu.VMEM((_LANES * E,), jnp.float32),
        ],
        compiler_params=pltpu.CompilerParams(needs_layout_passes=False),
    )(_combine_sc)
    flat = sc(route[:, 0], route[:, 1],
              router_weights[:, 0], router_weights[:, 1])
    return flat.reshape(T, E)


def _silu(g):
    return g * jax.nn.sigmoid(g)


def _moe_kernel(ct_ref, x_ref, w1_ref, w3_ref, w2_ref, out_ref):
    e = pl.program_id(0)

    @pl.when(e == 0)
    def _():
        out_ref[:, :] = jnp.zeros_like(out_ref)

    x = x_ref[:, :].astype(jnp.bfloat16)

    def mm(a, b):
        return jax.lax.dot_general(
            a, b.astype(jnp.bfloat16),
            (((1,), (1,)), ((), ())),
            preferred_element_type=jnp.float32)

    g = mm(x, w1_ref[0, 0])
    u = mm(x, w3_ref[0, 0])
    h = (_silu(g) * u).astype(jnp.bfloat16)
    y = mm(h, w2_ref[0])

    ids = lax.broadcasted_iota(jnp.int32, (1, E), 1)
    sel = jnp.where(ids == e, ct_ref[:, :], 0.0)
    combine = jnp.sum(sel, axis=1, keepdims=True)

    out_ref[:, :] += combine * y


@jax.jit
def kernel(hidden_states, expert_routing_table, router_weights, w13, w2):
    route = expert_routing_table.astype(jnp.int32)
    combine_te = _routing_combine(route, router_weights)
    w13r = w13.reshape(E, 2, F, D)

    out = pl.pallas_call(
        _moe_kernel,
        grid=(E,),
        in_specs=[
            pl.BlockSpec((T, E), lambda e: (0, 0)),
            pl.BlockSpec((T, D), lambda e: (0, 0)),
            pl.BlockSpec((1, 1, F, D), lambda e: (e, 0, 0, 0)),
            pl.BlockSpec((1, 1, F, D), lambda e: (e, 1, 0, 0)),
            pl.BlockSpec((1, D, F), lambda e: (e, 0, 0)),
        ],
        out_specs=pl.BlockSpec((T, D), lambda e: (0, 0)),
        out_shape=jax.ShapeDtypeStruct((T, D), jnp.float32),
        compiler_params=pltpu.CompilerParams(
            dimension_semantics=("arbitrary",),
        ),
    )(combine_te, hidden_states, w13r, w13r, w2)
    return out


# speedup vs baseline: 1.0071x; 1.0071x over previous
import functools

import jax
import jax.numpy as jnp
from jax import lax
from jax.experimental import pallas as pl
from jax.experimental.pallas import tpu as pltpu
from jax.experimental.pallas import tpu_sc as plsc

E = 16
K = 2
T = 128
D = 1024
F = 2048

_NC = 2
_LANES = 16
_CHUNKS = T // _LANES


def _combine_sc(rt_hbm, wt_hbm, out_hbm, rt_v, wt_v, acc_v, sem0, sem1):
    wid = lax.axis_index("s") * _NC + lax.axis_index("c")

    @pl.when(wid == 0)
    def _():
        cp0 = pltpu.make_async_copy(rt_hbm, rt_v, sem0)
        cp1 = pltpu.make_async_copy(wt_hbm, wt_v, sem1)
        cp0.start()
        cp1.start()
        for r in range(T * E // _LANES):
            acc_v[pl.ds(r * _LANES, _LANES)] = jnp.zeros((_LANES,), jnp.float32)
        cp0.wait()
        cp1.wait()
        for c in range(_CHUNKS):
            t_off = (lax.iota(jnp.int32, _LANES) + c * _LANES) * E
            for k in range(K):
                e_vec = rt_v[k, pl.ds(c * _LANES, _LANES)]
                w_vec = wt_v[k, pl.ds(c * _LANES, _LANES)]
                plsc.addupdate_scatter(acc_v, [t_off + e_vec], w_vec)
        pltpu.sync_copy(acc_v, out_hbm)


def _routing_combine(route, router_weights):
    sc = functools.partial(
        pl.kernel,
        out_type=jax.ShapeDtypeStruct((T * E,), jnp.float32),
        mesh=plsc.VectorSubcoreMesh(core_axis_name="c", subcore_axis_name="s"),
        scratch_types=[
            pltpu.VMEM((K, T), jnp.int32),
            pltpu.VMEM((K, T), jnp.float32),
            pltpu.VMEM((T * E,), jnp.float32),
            pltpu.SemaphoreType.DMA,
            pltpu.SemaphoreType.DMA,
        ],
        compiler_params=pltpu.CompilerParams(needs_layout_passes=False),
    )(_combine_sc)
    flat = sc(route.T, router_weights.T)
    return flat.reshape(T, E)


def _silu(g):
    return g * jax.nn.sigmoid(g)


def _moe_kernel(ct_ref, x_ref, w1_ref, w3_ref, w2_ref, out_ref):
    e = pl.program_id(0)

    @pl.when(e == 0)
    def _():
        out_ref[:, :] = jnp.zeros_like(out_ref)

    x = x_ref[:, :].astype(jnp.bfloat16)

    def mm(a, b):
        return jax.lax.dot_general(
            a, b.astype(jnp.bfloat16),
            (((1,), (1,)), ((), ())),
            preferred_element_type=jnp.float32)

    g = mm(x, w1_ref[0, 0])
    u = mm(x, w3_ref[0, 0])
    h = (_silu(g) * u).astype(jnp.bfloat16)
    y = mm(h, w2_ref[0])

    ids = lax.broadcasted_iota(jnp.int32, (1, E), 1)
    sel = jnp.where(ids == e, ct_ref[:, :], 0.0)
    combine = jnp.sum(sel, axis=1, keepdims=True)

    out_ref[:, :] += combine * y


@jax.jit
def kernel(hidden_states, expert_routing_table, router_weights, w13, w2):
    route = expert_routing_table.astype(jnp.int32)
    combine_te = _routing_combine(route, router_weights)
    w13r = w13.reshape(E, 2, F, D)

    out = pl.pallas_call(
        _moe_kernel,
        grid=(E,),
        in_specs=[
            pl.BlockSpec((T, E), lambda e: (0, 0)),
            pl.BlockSpec((T, D), lambda e: (0, 0)),
            pl.BlockSpec((1, 1, F, D), lambda e: (e, 0, 0, 0)),
            pl.BlockSpec((1, 1, F, D), lambda e: (e, 1, 0, 0)),
            pl.BlockSpec((1, D, F), lambda e: (e, 0, 0)),
        ],
        out_specs=pl.BlockSpec((T, D), lambda e: (0, 0)),
        out_shape=jax.ShapeDtypeStruct((T, D), jnp.float32),
        compiler_params=pltpu.CompilerParams(
            dimension_semantics=("arbitrary",),
        ),
    )(combine_te, hidden_states, w13r, w13r, w2)
    return out


# speedup vs baseline: 1.1506x; 1.1425x over previous
import functools

import jax
import jax.numpy as jnp
from jax.experimental import pallas as pl
from jax.experimental.pallas import tpu as pltpu

E = 16
K = 2
T = 128
D = 1024
F = 2048

EB = 1
NE = E // EB


def _silu(g):
    return g * jax.nn.sigmoid(g)


def _moe_kernel(route_ref, rw_ref, x_ref, w1_ref, w3_ref, w2_ref, out_ref):
    i = pl.program_id(0)

    @pl.when(i == 0)
    def _():
        out_ref[:, :] = jnp.zeros_like(out_ref)

    x = x_ref[:, :]

    def mm(a, b):
        return jax.lax.dot_general(
            a, b.astype(jnp.bfloat16),
            (((1,), (1,)), ((), ())),
            preferred_element_type=jnp.float32)

    for s in range(EB):
        e = i * EB + s
        g = mm(x, w1_ref[s, 0])
        u = mm(x, w3_ref[s, 0])
        h = (_silu(g) * u).astype(jnp.bfloat16)
        y = mm(h, w2_ref[s])

        sel = (route_ref[:, :] == e).astype(jnp.float32)
        combine = jnp.sum(sel * rw_ref[:, :], axis=1, keepdims=True)

        out_ref[:, :] += combine * y


@jax.jit
def kernel(hidden_states, expert_routing_table, router_weights, w13, w2):
    route = expert_routing_table.astype(jnp.int32)
    xb = hidden_states.astype(jnp.bfloat16)
    w13r = w13.reshape(E, 2, F, D)

    out = pl.pallas_call(
        _moe_kernel,
        grid=(NE,),
        in_specs=[
            pl.BlockSpec((T, K), lambda i: (0, 0)),
            pl.BlockSpec((T, K), lambda i: (0, 0)),
            pl.BlockSpec((T, D), lambda i: (0, 0)),
            pl.BlockSpec((EB, 1, F, D), lambda i: (i, 0, 0, 0)),
            pl.BlockSpec((EB, 1, F, D), lambda i: (i, 1, 0, 0)),
            pl.BlockSpec((EB, D, F), lambda i: (i, 0, 0)),
        ],
        out_specs=pl.BlockSpec((T, D), lambda i: (0, 0)),
        out_shape=jax.ShapeDtypeStruct((T, D), jnp.float32),
        compiler_params=pltpu.CompilerParams(
            dimension_semantics=("arbitrary",),
            vmem_limit_bytes=60 * 1024 * 1024,
        ),
    )(route, router_weights, xb, w13r, w13r, w2)
    return out


# speedup vs baseline: 1.1737x; 1.0201x over previous
import functools

import jax
import jax.numpy as jnp
from jax.experimental import pallas as pl
from jax.experimental.pallas import tpu as pltpu

E = 16
K = 2
T = 128
D = 1024
F = 2048

EB = 1
NE = E // EB


def _silu(g):
    return g * jax.nn.sigmoid(g)


def _moe_kernel(route_ref, rw_ref, x_ref, w1_ref, w3_ref, w2_ref, out_ref):
    i = pl.program_id(0)

    @pl.when(i == 0)
    def _():
        out_ref[:, :] = jnp.zeros_like(out_ref)

    x = x_ref[:, :].astype(jnp.bfloat16)

    def mm(a, b):
        return jax.lax.dot_general(
            a, b.astype(jnp.bfloat16),
            (((1,), (1,)), ((), ())),
            preferred_element_type=jnp.float32)

    for s in range(EB):
        e = i * EB + s
        g = mm(x, w1_ref[s, 0])
        u = mm(x, w3_ref[s, 0])
        h = (_silu(g) * u).astype(jnp.bfloat16)
        y = mm(h, w2_ref[s])

        sel = (route_ref[:, :] == e).astype(jnp.float32)
        combine = jnp.sum(sel * rw_ref[:, :], axis=1, keepdims=True)

        out_ref[:, :] += combine * y


@jax.jit
def kernel(hidden_states, expert_routing_table, router_weights, w13, w2):
    route = expert_routing_table.astype(jnp.int32)
    w13r = w13.reshape(E, 2, F, D)

    out = pl.pallas_call(
        _moe_kernel,
        grid=(NE,),
        in_specs=[
            pl.BlockSpec((T, K), lambda i: (0, 0)),
            pl.BlockSpec((T, K), lambda i: (0, 0)),
            pl.BlockSpec((T, D), lambda i: (0, 0)),
            pl.BlockSpec((EB, 1, F, D), lambda i: (i, 0, 0, 0)),
            pl.BlockSpec((EB, 1, F, D), lambda i: (i, 1, 0, 0)),
            pl.BlockSpec((EB, D, F), lambda i: (i, 0, 0)),
        ],
        out_specs=pl.BlockSpec((T, D), lambda i: (0, 0)),
        out_shape=jax.ShapeDtypeStruct((T, D), jnp.float32),
        compiler_params=pltpu.CompilerParams(
            dimension_semantics=("arbitrary",),
            vmem_limit_bytes=60 * 1024 * 1024,
        ),
    )(route, router_weights, hidden_states, w13r, w13r, w2)
    return out
